# Initial kernel scaffold; baseline (speedup 1.0000x reference)
import jax
import jax.numpy as jnp
from jax.experimental import pallas as pl


def kernel(x, edge_index, edge_attr, batch, multihop_edge_index, distance, W, lin_b, bn_gamma, bn_beta, bn_mean, bn_var):
    raise NotImplementedError("write your pallas kernel here")


# speedup vs baseline: 22.6248x; 22.6248x over previous
import jax
import jax.numpy as jnp
from jax import lax
from jax.experimental import pallas as pl
from jax.experimental.pallas import tpu as pltpu
from jax.experimental.pallas import tpu_sc as plsc

N = 10000
E = 320000
D = 128
N_PAD = 10240
NC, NS = 2, 16
NW = NC * NS
K = 80
EPT = E // NW
NCH = EPT // K
SPT = N_PAD // NS
RB = 1024


def _hist_body(col3_hbm, deg_hbm, idxb, rows, sh_deg):
    c = lax.axis_index("c")
    s = lax.axis_index("s")
    wid = s * NC + c
    z16 = jnp.zeros((16,), jnp.float32)
    one16 = jnp.full((16,), 1.0, jnp.float32)

    @pl.loop(0, (K * D) // 16)
    def _zero(i):
        rows[i >> 3, pl.ds((i & 7) * 16, 16)] = z16

    for b in range(8):
        pltpu.sync_copy(rows, sh_deg.at[pl.ds(s * SPT + b * K, K), :])

    @pl.loop(0, (K * D) // 16)
    def _ones(i):
        rows[i >> 3, pl.ds((i & 7) * 16, 16)] = one16

    pltpu.sync_copy(col3_hbm.at[wid], idxb)

    plsc.subcore_barrier()

    @pl.loop(0, NCH)
    def _count(j):
        pltpu.sync_copy(rows, sh_deg.at[idxb.at[j]], add=True)

    plsc.subcore_barrier()

    for b in range(8):
        pltpu.sync_copy(sh_deg.at[pl.ds(s * SPT + b * K, K), :], rows)
        pltpu.sync_copy(rows, deg_hbm.at[c, pl.ds(s * SPT + b * K, K), :])


def _msg_body(y_hbm, row3_hbm, col3_hbm, out_hbm, idx_r, idx_c, rows, sem,
              sh_acc):
    c = lax.axis_index("c")
    s = lax.axis_index("s")
    wid = s * NC + c
    z16 = jnp.zeros((16,), jnp.float32)

    @pl.loop(0, (K * D) // 16)
    def _zero(i):
        rows[i >> 3, pl.ds((i & 7) * 16, 16)] = z16

    for b in range(8):
        pltpu.sync_copy(rows, sh_acc.at[pl.ds(s * SPT + b * K, K), :])

    pltpu.sync_copy(row3_hbm.at[wid], idx_r)
    pltpu.sync_copy(col3_hbm.at[wid], idx_c)

    plsc.subcore_barrier()

    @pl.loop(0, NCH)
    def _edges(j):
        pltpu.async_copy(y_hbm.at[idx_r.at[j]], rows, sem).wait()
        pltpu.sync_copy(rows, sh_acc.at[idx_c.at[j]], add=True)

    plsc.subcore_barrier()

    for b in range(8):
        pltpu.sync_copy(sh_acc.at[pl.ds(s * SPT + b * K, K), :], rows)
        pltpu.sync_copy(rows, out_hbm.at[c, pl.ds(s * SPT + b * K, K), :])


def _mm_body(x_ref, w_ref, dg_ref, y_ref):
    xw = lax.dot_general(x_ref[...], w_ref[...], (((1,), (1,)), ((), ())),
                         preferred_element_type=jnp.float32)
    deg = dg_ref[0, :, 0:1] + dg_ref[1, :, 0:1] + 1.0
    y_ref[...] = xw * lax.rsqrt(deg)


def _epi_body(p_ref, y_ref, dg_ref, b_ref, g_ref, bt_ref, m_ref, v_ref,
              o_ref):
    acc = p_ref[0] + p_ref[1] + y_ref[...]
    deg = dg_ref[0, :, 0:1] + dg_ref[1, :, 0:1] + 1.0
    h = acc * lax.rsqrt(deg) + b_ref[...]
    a = g_ref[...] * lax.rsqrt(v_ref[...] + 1e-5)
    h = (h - m_ref[...]) * a + bt_ref[...]
    o_ref[...] = jnp.maximum(h, 0.0)


def kernel(x, edge_index, edge_attr, batch, multihop_edge_index, distance,
           W, lin_b, bn_gamma, bn_beta, bn_mean, bn_var):
    del edge_attr, batch, multihop_edge_index, distance

    mesh = plsc.VectorSubcoreMesh(core_axis_name="c", subcore_axis_name="s")

    hist = pl.kernel(
        _hist_body,
        out_type=jax.ShapeDtypeStruct((NC, N_PAD, D), jnp.float32),
        mesh=mesh,
        scratch_types=[
            pltpu.VMEM((NCH, K), jnp.int32),
            pltpu.VMEM((K, D), jnp.float32),
            pltpu.VMEM_SHARED((N_PAD, D), jnp.float32),
        ],
    )

    msg = pl.kernel(
        _msg_body,
        out_type=jax.ShapeDtypeStruct((NC, N_PAD, D), jnp.float32),
        mesh=mesh,
        scratch_types=[
            pltpu.VMEM((NCH, K), jnp.int32),
            pltpu.VMEM((NCH, K), jnp.int32),
            pltpu.VMEM((K, D), jnp.float32),
            pltpu.SemaphoreType.DMA,
            pltpu.VMEM_SHARED((N_PAD, D), jnp.float32),
        ],
    )

    grid = N_PAD // RB
    mm = pl.pallas_call(
        _mm_body,
        grid=(grid,),
        in_specs=[
            pl.BlockSpec((RB, D), lambda i: (i, 0)),
            pl.BlockSpec((D, D), lambda i: (0, 0)),
            pl.BlockSpec((NC, RB, D), lambda i: (0, i, 0)),
        ],
        out_specs=pl.BlockSpec((RB, D), lambda i: (i, 0)),
        out_shape=jax.ShapeDtypeStruct((N_PAD, D), jnp.float32),
    )

    epi = pl.pallas_call(
        _epi_body,
        grid=(grid,),
        in_specs=[
            pl.BlockSpec((NC, RB, D), lambda i: (0, i, 0)),
            pl.BlockSpec((RB, D), lambda i: (i, 0)),
            pl.BlockSpec((NC, RB, D), lambda i: (0, i, 0)),
        ] + [pl.BlockSpec((1, D), lambda i: (0, 0))] * 5,
        out_specs=pl.BlockSpec((RB, D), lambda i: (i, 0)),
        out_shape=jax.ShapeDtypeStruct((N_PAD, D), jnp.float32),
    )

    x_pad = jnp.pad(x, ((0, N_PAD - N), (0, 0)))
    row3 = edge_index[0].reshape(NW, NCH, K)
    col3 = edge_index[1].reshape(NW, NCH, K)

    deg_p = hist(col3)
    y = mm(x_pad, W, deg_p)
    partial = msg(y, row3, col3)
    out = epi(partial, y, deg_p,
              lin_b.reshape(1, D), bn_gamma.reshape(1, D),
              bn_beta.reshape(1, D), bn_mean.reshape(1, D),
              bn_var.reshape(1, D))
    return out[:N]
